# Initial kernel scaffold; baseline (speedup 1.0000x reference)
import jax
import jax.numpy as jnp
from jax.experimental import pallas as pl


def kernel(x, edge_index, neg_edge_index, eps, W1, b1, Wmu, bmu, Wlv, blv):
    raise NotImplementedError("write your pallas kernel here")


# speedup vs baseline: 7.4177x; 7.4177x over previous
import jax
import jax.numpy as jnp
from jax import lax
from jax.experimental import pallas as pl
from jax.experimental.pallas import tpu as pltpu
from jax.experimental.pallas import tpu_sc as plsc

N = 10000
E = 320000
IN_DIM = 128
HID = 64
ZD = 32

NC, NS, L = 2, 16, 16
NW = NC * NS
EP = E // NW
EPD = 2 * E // NW
K = 80
NPAD = 10240
NPT = NPAD // NS

RB = 1000
GRID = N // RB

_MESH = plsc.VectorSubcoreMesh(
    core_axis_name="c", subcore_axis_name="s", num_cores=NC, num_subcores=NS)
_SC_PARAMS = pltpu.CompilerParams(needs_layout_passes=False,
                                  use_tc_tiling_on_sc=False)


def _sc_deg_body(dst_hbm, out_hbm, dstbuf, countbuf):
    cid = lax.axis_index("c")
    sid = lax.axis_index("s")
    wid = sid * NC + cid
    zeros16 = jnp.zeros((L,), jnp.float32)
    ones16 = jnp.ones((L,), jnp.float32)

    def zb(i, c):
        countbuf[pl.ds(i * L, L)] = zeros16
        return c
    lax.fori_loop(0, N // L, zb, 0)

    pltpu.sync_copy(dst_hbm.at[pl.ds(wid * EP, EP)], dstbuf)

    def cb(i, c):
        idx = dstbuf[pl.ds(i * L, L)]
        plsc.addupdate_scatter(countbuf, [idx], ones16)
        return c
    lax.fori_loop(0, EP // L, cb, 0)

    for g in range(GRID):
        pltpu.sync_copy(countbuf.at[pl.ds(g * RB, RB)], out_hbm.at[g, wid])


_deg_call = pl.kernel(
    _sc_deg_body,
    out_type=jax.ShapeDtypeStruct((GRID, NW, RB), jnp.float32),
    mesh=_MESH,
    compiler_params=_SC_PARAMS,
    scratch_types=[
        pltpu.VMEM((EP,), jnp.int32),
        pltpu.VMEM((N,), jnp.float32),
    ],
)


def _sc_agg_body(tab_hbm, src_hbm, dst_hbm, out_hbm, sidx, didx, rows, zrow,
                 acc, sem):
    cid = lax.axis_index("c")
    sid = lax.axis_index("s")
    wid = sid * NC + cid
    zeros16 = jnp.zeros((L,), jnp.float32)

    def zb(i, c):
        for j in range(HID // L):
            zrow[i, pl.ds(j * L, L)] = zeros16
        return c
    lax.fori_loop(0, NPT, zb, 0)
    pltpu.sync_copy(zrow, acc.at[pl.ds(sid * NPT, NPT)])
    plsc.subcore_barrier()

    def cb(i, c):
        base = wid * EP + i * K
        pltpu.sync_copy(src_hbm.at[pl.ds(base, K)], sidx)
        pltpu.sync_copy(dst_hbm.at[pl.ds(base, K)], didx)
        pltpu.async_copy(tab_hbm.at[sidx], rows, sem).wait()
        pltpu.sync_copy(rows, acc.at[didx], add=True)
        return c
    lax.fori_loop(0, EP // K, cb, 0)

    plsc.subcore_barrier()
    pltpu.sync_copy(acc.at[pl.ds(sid * NPT, NPT)],
                    out_hbm.at[cid, pl.ds(sid * NPT, NPT)])


_agg_call = pl.kernel(
    _sc_agg_body,
    out_type=jax.ShapeDtypeStruct((NC, NPAD, HID), jnp.float32),
    mesh=_MESH,
    compiler_params=_SC_PARAMS,
    scratch_types=[
        pltpu.VMEM((K,), jnp.int32),
        pltpu.VMEM((K,), jnp.int32),
        pltpu.VMEM((K, HID), jnp.float32),
        pltpu.VMEM((NPT, HID), jnp.float32),
        pltpu.VMEM_SHARED((NPAD, HID), jnp.float32),
        pltpu.SemaphoreType.DMA,
    ],
)


def _sc_dec_body(z_hbm, src_hbm, dst_hbm, out_hbm, sidx, didx, zs, zd, lbuf,
                 sem):
    cid = lax.axis_index("c")
    sid = lax.axis_index("s")
    wid = sid * NC + cid
    iota = lax.iota(jnp.int32, L)

    def cb(i, c):
        base = wid * EPD + i * K
        pltpu.sync_copy(src_hbm.at[pl.ds(base, K)], sidx)
        pltpu.sync_copy(dst_hbm.at[pl.ds(base, K)], didx)
        pltpu.async_copy(z_hbm.at[sidx], zs, sem).wait()
        pltpu.async_copy(z_hbm.at[didx], zd, sem).wait()
        for g in range(K // L):
            rows = iota + (g * L)
            acc16 = jnp.zeros((L,), jnp.float32)
            for j in range(ZD):
                col = jnp.full((L,), j, jnp.int32)
                a = plsc.load_gather(zs, [rows, col])
                b = plsc.load_gather(zd, [rows, col])
                acc16 = acc16 + a * b
            lbuf[pl.ds(g * L, L)] = acc16
        pltpu.sync_copy(lbuf, out_hbm.at[pl.ds(base, K)])
        return c
    lax.fori_loop(0, EPD // K, cb, 0)


_dec_call = pl.kernel(
    _sc_dec_body,
    out_type=jax.ShapeDtypeStruct((2 * E,), jnp.float32),
    mesh=_MESH,
    compiler_params=_SC_PARAMS,
    scratch_types=[
        pltpu.VMEM((K,), jnp.int32),
        pltpu.VMEM((K,), jnp.int32),
        pltpu.VMEM((K, ZD), jnp.float32),
        pltpu.VMEM((K, ZD), jnp.float32),
        pltpu.VMEM((K,), jnp.float32),
        pltpu.SemaphoreType.DMA,
    ],
)


def _tc_prep_body(counts_ref, x_ref, w1_ref, hs_ref, dinv_ref):
    deg = jnp.sum(counts_ref[0], axis=0) + 1.0
    dinv = lax.rsqrt(deg)
    h = jnp.dot(x_ref[...], w1_ref[...], preferred_element_type=jnp.float32)
    hs_ref[...] = h * dinv[:, None]
    dinv_ref[...] = dinv[:, None]


def _tc_h_body(t_ref, hs_ref, dinv_ref, b1_ref, out_ref):
    t = t_ref[0] + t_ref[1] + hs_ref[...]
    dinv = dinv_ref[...]
    h = jnp.maximum(t * dinv + b1_ref[...], 0.0)
    out_ref[...] = h * dinv


def _tc_z_body(t_ref, hs2_ref, dinv_ref, eps_ref, wmu_ref, bmu_ref, wlv_ref,
               blv_ref, z_ref, kl_ref):
    i = pl.program_id(0)
    agg = (t_ref[0] + t_ref[1] + hs2_ref[...]) * dinv_ref[...]
    mu = jnp.dot(agg, wmu_ref[...], preferred_element_type=jnp.float32) + bmu_ref[...]
    lv = jnp.dot(agg, wlv_ref[...], preferred_element_type=jnp.float32) + blv_ref[...]
    z_ref[...] = mu + eps_ref[...] * jnp.exp(0.5 * lv)
    klp = jnp.sum(1.0 + lv - mu * mu - jnp.exp(lv)).reshape(1, 1)

    @pl.when(i == 0)
    def _():
        kl_ref[...] = klp

    @pl.when(i > 0)
    def _():
        kl_ref[...] = kl_ref[...] + klp


def _tc_loss_body(lp_ref, ln_ref, kl_ref, loss_ref, recon_ref, klo_ref):
    lp = lp_ref[...]
    ln = ln_ref[...]
    sp_pos = jnp.maximum(lp, 0.0) - lp + jnp.log1p(jnp.exp(-jnp.abs(lp)))
    sp_neg = jnp.maximum(ln, 0.0) + jnp.log1p(jnp.exp(-jnp.abs(ln)))
    recon = ((jnp.sum(sp_pos) + jnp.sum(sp_neg)) / (2.0 * E)).reshape(1, 1)
    kl = -0.5 * kl_ref[...] / (N * ZD)
    loss_ref[...] = recon + kl
    recon_ref[...] = recon
    klo_ref[...] = kl


def kernel(x, edge_index, neg_edge_index, eps, W1, b1, Wmu, bmu, Wlv, blv):
    src = edge_index[0].astype(jnp.int32)
    dst = edge_index[1].astype(jnp.int32)

    counts = _deg_call(dst)

    hs1, dinv = pl.pallas_call(
        _tc_prep_body,
        grid=(GRID,),
        in_specs=[
            pl.BlockSpec((1, NW, RB), lambda i: (i, 0, 0)),
            pl.BlockSpec((RB, IN_DIM), lambda i: (i, 0)),
            pl.BlockSpec((IN_DIM, HID), lambda i: (0, 0)),
        ],
        out_specs=[
            pl.BlockSpec((RB, HID), lambda i: (i, 0)),
            pl.BlockSpec((RB, 1), lambda i: (i, 0)),
        ],
        out_shape=[
            jax.ShapeDtypeStruct((N, HID), jnp.float32),
            jax.ShapeDtypeStruct((N, 1), jnp.float32),
        ],
    )(counts, x, W1)

    t1 = _agg_call(hs1, src, dst)

    hs2 = pl.pallas_call(
        _tc_h_body,
        grid=(GRID,),
        in_specs=[
            pl.BlockSpec((NC, RB, HID), lambda i: (0, i, 0)),
            pl.BlockSpec((RB, HID), lambda i: (i, 0)),
            pl.BlockSpec((RB, 1), lambda i: (i, 0)),
            pl.BlockSpec((1, HID), lambda i: (0, 0)),
        ],
        out_specs=pl.BlockSpec((RB, HID), lambda i: (i, 0)),
        out_shape=jax.ShapeDtypeStruct((N, HID), jnp.float32),
    )(t1, hs1, dinv, b1.reshape(1, HID))

    t2 = _agg_call(hs2, src, dst)

    z, klsum = pl.pallas_call(
        _tc_z_body,
        grid=(GRID,),
        in_specs=[
            pl.BlockSpec((NC, RB, HID), lambda i: (0, i, 0)),
            pl.BlockSpec((RB, HID), lambda i: (i, 0)),
            pl.BlockSpec((RB, 1), lambda i: (i, 0)),
            pl.BlockSpec((RB, ZD), lambda i: (i, 0)),
            pl.BlockSpec((HID, ZD), lambda i: (0, 0)),
            pl.BlockSpec((1, ZD), lambda i: (0, 0)),
            pl.BlockSpec((HID, ZD), lambda i: (0, 0)),
            pl.BlockSpec((1, ZD), lambda i: (0, 0)),
        ],
        out_specs=[
            pl.BlockSpec((RB, ZD), lambda i: (i, 0)),
            pl.BlockSpec((1, 1), lambda i: (0, 0)),
        ],
        out_shape=[
            jax.ShapeDtypeStruct((N, ZD), jnp.float32),
            jax.ShapeDtypeStruct((1, 1), jnp.float32),
        ],
    )(t2, hs2, dinv, eps, Wmu, bmu.reshape(1, ZD), Wlv, blv.reshape(1, ZD))

    src_all = jnp.concatenate([src, neg_edge_index[0].astype(jnp.int32)])
    dst_all = jnp.concatenate([dst, neg_edge_index[1].astype(jnp.int32)])
    logits = _dec_call(z, src_all, dst_all)

    lp = logits[:E].reshape(E // 128, 128)
    ln = logits[E:].reshape(E // 128, 128)

    loss, recon, kl = pl.pallas_call(
        _tc_loss_body,
        in_specs=[
            pl.BlockSpec((E // 128, 128), lambda: (0, 0)),
            pl.BlockSpec((E // 128, 128), lambda: (0, 0)),
            pl.BlockSpec((1, 1), lambda: (0, 0)),
        ],
        out_specs=[
            pl.BlockSpec((1, 1), lambda: (0, 0)),
            pl.BlockSpec((1, 1), lambda: (0, 0)),
            pl.BlockSpec((1, 1), lambda: (0, 0)),
        ],
        out_shape=[
            jax.ShapeDtypeStruct((1, 1), jnp.float32),
            jax.ShapeDtypeStruct((1, 1), jnp.float32),
            jax.ShapeDtypeStruct((1, 1), jnp.float32),
        ],
    )(lp, ln, klsum)

    return (loss.reshape(()),
            jax.lax.stop_gradient(recon.reshape(())),
            jax.lax.stop_gradient(kl.reshape(())))


# speedup vs baseline: 16.7832x; 2.2626x over previous
import jax
import jax.numpy as jnp
from jax import lax
from jax.experimental import pallas as pl
from jax.experimental.pallas import tpu as pltpu
from jax.experimental.pallas import tpu_sc as plsc

N = 10000
E = 320000
IN_DIM = 128
HID = 64
ZD = 32

NC, NS, L = 2, 16, 16
NW = NC * NS
EP = E // NW
EPD = 2 * E // NW
SUBK = 80
NSUBC = 5
CHE = SUBK * NSUBC
NCH_A = EP // CHE
NCH_D = EPD // CHE
NPAD = 10240
NPT = NPAD // NS
ZROWS = 160

RB = 1000
GRID = N // RB

_MESH = plsc.VectorSubcoreMesh(
    core_axis_name="c", subcore_axis_name="s", num_cores=NC, num_subcores=NS)
_SC_PARAMS = pltpu.CompilerParams(needs_layout_passes=False,
                                  use_tc_tiling_on_sc=False)


def _sc_deg_body(dst_hbm, out_hbm, dstbuf, countbuf):
    cid = lax.axis_index("c")
    sid = lax.axis_index("s")
    wid = sid * NC + cid
    zeros16 = jnp.zeros((L,), jnp.float32)
    ones16 = jnp.ones((L,), jnp.float32)

    def zb(i, c):
        countbuf[pl.ds(i * L, L)] = zeros16
        return c
    lax.fori_loop(0, N // L, zb, 0)

    pltpu.sync_copy(dst_hbm.at[pl.ds(wid * EP, EP)], dstbuf)

    def cb(i, c):
        idx = dstbuf[pl.ds(i * L, L)]
        plsc.addupdate_scatter(countbuf, [idx], ones16)
        return c
    lax.fori_loop(0, EP // L, cb, 0)

    for g in range(GRID):
        pltpu.sync_copy(countbuf.at[pl.ds(g * RB, RB)], out_hbm.at[g, wid])


_deg_call = pl.kernel(
    _sc_deg_body,
    out_type=jax.ShapeDtypeStruct((GRID, NW, RB), jnp.float32),
    mesh=_MESH,
    compiler_params=_SC_PARAMS,
    scratch_types=[
        pltpu.VMEM((EP,), jnp.int32),
        pltpu.VMEM((N,), jnp.float32),
    ],
)


def _sc_agg_body(tab_hbm, src_hbm, dst3_hbm, out_hbm, sidx, didx2, rows, zbuf,
                 acc, gsem, ssem):
    cid = lax.axis_index("c")
    sid = lax.axis_index("s")
    wid = sid * NC + cid
    zeros16 = jnp.zeros((L,), jnp.float32)

    def zb(i, c):
        for j in range(HID // L):
            zbuf[i, pl.ds(j * L, L)] = zeros16
        return c
    lax.fori_loop(0, ZROWS, zb, 0)
    for r in range(NPT // ZROWS):
        pltpu.sync_copy(zbuf, acc.at[pl.ds(sid * NPT + r * ZROWS, ZROWS)])
    plsc.subcore_barrier()

    pltpu.sync_copy(src_hbm.at[pl.ds(wid * EP, EP)], sidx)
    pltpu.sync_copy(dst3_hbm.at[wid], didx2)

    def start_gathers(c, b):
        for j in range(NSUBC):
            pltpu.async_copy(
                tab_hbm.at[sidx.at[pl.ds(c * CHE + j * SUBK, SUBK)]],
                rows.at[b].at[pl.ds(j * SUBK, SUBK)], gsem.at[b])

    start_gathers(0, 0)
    start_gathers(1, 1)

    def body(c, carry):
        b = lax.rem(c, 2)
        pltpu.make_async_copy(tab_hbm.at[pl.ds(0, CHE)], rows.at[b],
                              gsem.at[b]).wait()
        for j in range(NSUBC):
            pltpu.async_copy(rows.at[b].at[pl.ds(j * SUBK, SUBK)],
                             acc.at[didx2.at[c * NSUBC + j]], ssem.at[b],
                             add=True)
        pltpu.make_async_copy(tab_hbm.at[pl.ds(0, CHE)], rows.at[b],
                              ssem.at[b]).wait()

        @pl.when(c + 2 < NCH_A)
        def _():
            start_gathers(c + 2, b)
        return carry
    lax.fori_loop(0, NCH_A, body, 0)

    plsc.subcore_barrier()
    pltpu.sync_copy(acc.at[pl.ds(sid * NPT, NPT)],
                    out_hbm.at[cid, pl.ds(sid * NPT, NPT)])


_agg_call = pl.kernel(
    _sc_agg_body,
    out_type=jax.ShapeDtypeStruct((NC, NPAD, HID), jnp.float32),
    mesh=_MESH,
    compiler_params=_SC_PARAMS,
    scratch_types=[
        pltpu.VMEM((EP,), jnp.int32),
        pltpu.VMEM((NCH_A * NSUBC, SUBK), jnp.int32),
        pltpu.VMEM((2, CHE, HID), jnp.float32),
        pltpu.VMEM((ZROWS, HID), jnp.float32),
        pltpu.VMEM_SHARED((NPAD, HID), jnp.float32),
        pltpu.SemaphoreType.DMA((2,)),
        pltpu.SemaphoreType.DMA((2,)),
    ],
)


def _sc_dec_body(z_hbm, src_hbm, dst_hbm, out_hbm, sidx, didx, zs, zd, lbuf,
                 gsem):
    cid = lax.axis_index("c")
    sid = lax.axis_index("s")
    wid = sid * NC + cid
    iota = lax.iota(jnp.int32, L)
    zeros16 = jnp.zeros((L,), jnp.float32)

    pltpu.sync_copy(src_hbm.at[pl.ds(wid * EPD, EPD)], sidx)
    pltpu.sync_copy(dst_hbm.at[pl.ds(wid * EPD, EPD)], didx)

    def start_gathers(c, b):
        for j in range(NSUBC):
            off = c * CHE + j * SUBK
            pltpu.async_copy(z_hbm.at[sidx.at[pl.ds(off, SUBK)]],
                             zs.at[b].at[pl.ds(j * SUBK, SUBK)], gsem.at[b])
            pltpu.async_copy(z_hbm.at[didx.at[pl.ds(off, SUBK)]],
                             zd.at[b].at[pl.ds(j * SUBK, SUBK)], gsem.at[b])

    start_gathers(0, 0)
    start_gathers(1, 1)

    def body(c, carry):
        b = lax.rem(c, 2)
        pltpu.make_async_copy(z_hbm.at[pl.ds(0, CHE)], zs.at[b],
                              gsem.at[b]).wait()
        pltpu.make_async_copy(z_hbm.at[pl.ds(0, CHE)], zd.at[b],
                              gsem.at[b]).wait()
        bvec = jnp.full((L,), b, jnp.int32)

        def gb(g, cc):
            rows16 = iota + g * L
            acc16 = zeros16
            for j in range(ZD):
                col = jnp.full((L,), j, jnp.int32)
                a = plsc.load_gather(zs, [bvec, rows16, col])
                d = plsc.load_gather(zd, [bvec, rows16, col])
                acc16 = acc16 + a * d
            lbuf[pl.ds(c * CHE + g * L, L)] = acc16
            return cc
        lax.fori_loop(0, CHE // L, gb, 0)

        @pl.when(c + 2 < NCH_D)
        def _():
            start_gathers(c + 2, b)
        return carry
    lax.fori_loop(0, NCH_D, body, 0)

    pltpu.sync_copy(lbuf, out_hbm.at[pl.ds(wid * EPD, EPD)])


_dec_call = pl.kernel(
    _sc_dec_body,
    out_type=jax.ShapeDtypeStruct((2 * E,), jnp.float32),
    mesh=_MESH,
    compiler_params=_SC_PARAMS,
    scratch_types=[
        pltpu.VMEM((EPD,), jnp.int32),
        pltpu.VMEM((EPD,), jnp.int32),
        pltpu.VMEM((2, CHE, ZD), jnp.float32),
        pltpu.VMEM((2, CHE, ZD), jnp.float32),
        pltpu.VMEM((EPD,), jnp.float32),
        pltpu.SemaphoreType.DMA((2,)),
    ],
)


def _tc_prep_body(counts_ref, x_ref, w1_ref, hs_ref, dinv_ref):
    deg = jnp.sum(counts_ref[0], axis=0) + 1.0
    dinv = lax.rsqrt(deg)
    h = jnp.dot(x_ref[...], w1_ref[...], preferred_element_type=jnp.float32)
    hs_ref[...] = h * dinv[:, None]
    dinv_ref[...] = dinv[:, None]


def _tc_h_body(t_ref, hs_ref, dinv_ref, b1_ref, out_ref):
    t = t_ref[0] + t_ref[1] + hs_ref[...]
    dinv = dinv_ref[...]
    h = jnp.maximum(t * dinv + b1_ref[...], 0.0)
    out_ref[...] = h * dinv


def _tc_z_body(t_ref, hs2_ref, dinv_ref, eps_ref, wmu_ref, bmu_ref, wlv_ref,
               blv_ref, z_ref, kl_ref):
    i = pl.program_id(0)
    agg = (t_ref[0] + t_ref[1] + hs2_ref[...]) * dinv_ref[...]
    mu = jnp.dot(agg, wmu_ref[...], preferred_element_type=jnp.float32) + bmu_ref[...]
    lv = jnp.dot(agg, wlv_ref[...], preferred_element_type=jnp.float32) + blv_ref[...]
    z_ref[...] = mu + eps_ref[...] * jnp.exp(0.5 * lv)
    klp = jnp.sum(1.0 + lv - mu * mu - jnp.exp(lv)).reshape(1, 1)

    @pl.when(i == 0)
    def _():
        kl_ref[...] = klp

    @pl.when(i > 0)
    def _():
        kl_ref[...] = kl_ref[...] + klp


def _tc_loss_body(lp_ref, ln_ref, kl_ref, loss_ref, recon_ref, klo_ref):
    lp = lp_ref[...]
    ln = ln_ref[...]
    sp_pos = jnp.maximum(lp, 0.0) - lp + jnp.log1p(jnp.exp(-jnp.abs(lp)))
    sp_neg = jnp.maximum(ln, 0.0) + jnp.log1p(jnp.exp(-jnp.abs(ln)))
    recon = ((jnp.sum(sp_pos) + jnp.sum(sp_neg)) / (2.0 * E)).reshape(1, 1)
    kl = -0.5 * kl_ref[...] / (N * ZD)
    loss_ref[...] = recon + kl
    recon_ref[...] = recon
    klo_ref[...] = kl


def kernel(x, edge_index, neg_edge_index, eps, W1, b1, Wmu, bmu, Wlv, blv):
    src = edge_index[0].astype(jnp.int32)
    dst = edge_index[1].astype(jnp.int32)
    dst3 = dst.reshape(NW, NCH_A * NSUBC, SUBK)

    counts = _deg_call(dst)

    hs1, dinv = pl.pallas_call(
        _tc_prep_body,
        grid=(GRID,),
        in_specs=[
            pl.BlockSpec((1, NW, RB), lambda i: (i, 0, 0)),
            pl.BlockSpec((RB, IN_DIM), lambda i: (i, 0)),
            pl.BlockSpec((IN_DIM, HID), lambda i: (0, 0)),
        ],
        out_specs=[
            pl.BlockSpec((RB, HID), lambda i: (i, 0)),
            pl.BlockSpec((RB, 1), lambda i: (i, 0)),
        ],
        out_shape=[
            jax.ShapeDtypeStruct((N, HID), jnp.float32),
            jax.ShapeDtypeStruct((N, 1), jnp.float32),
        ],
    )(counts, x, W1)

    t1 = _agg_call(hs1, src, dst3)

    hs2 = pl.pallas_call(
        _tc_h_body,
        grid=(GRID,),
        in_specs=[
            pl.BlockSpec((NC, RB, HID), lambda i: (0, i, 0)),
            pl.BlockSpec((RB, HID), lambda i: (i, 0)),
            pl.BlockSpec((RB, 1), lambda i: (i, 0)),
            pl.BlockSpec((1, HID), lambda i: (0, 0)),
        ],
        out_specs=pl.BlockSpec((RB, HID), lambda i: (i, 0)),
        out_shape=jax.ShapeDtypeStruct((N, HID), jnp.float32),
    )(t1, hs1, dinv, b1.reshape(1, HID))

    t2 = _agg_call(hs2, src, dst3)

    z, klsum = pl.pallas_call(
        _tc_z_body,
        grid=(GRID,),
        in_specs=[
            pl.BlockSpec((NC, RB, HID), lambda i: (0, i, 0)),
            pl.BlockSpec((RB, HID), lambda i: (i, 0)),
            pl.BlockSpec((RB, 1), lambda i: (i, 0)),
            pl.BlockSpec((RB, ZD), lambda i: (i, 0)),
            pl.BlockSpec((HID, ZD), lambda i: (0, 0)),
            pl.BlockSpec((1, ZD), lambda i: (0, 0)),
            pl.BlockSpec((HID, ZD), lambda i: (0, 0)),
            pl.BlockSpec((1, ZD), lambda i: (0, 0)),
        ],
        out_specs=[
            pl.BlockSpec((RB, ZD), lambda i: (i, 0)),
            pl.BlockSpec((1, 1), lambda i: (0, 0)),
        ],
        out_shape=[
            jax.ShapeDtypeStruct((N, ZD), jnp.float32),
            jax.ShapeDtypeStruct((1, 1), jnp.float32),
        ],
    )(t2, hs2, dinv, eps, Wmu, bmu.reshape(1, ZD), Wlv, blv.reshape(1, ZD))

    src_all = jnp.concatenate([src, neg_edge_index[0].astype(jnp.int32)])
    dst_all = jnp.concatenate([dst, neg_edge_index[1].astype(jnp.int32)])
    logits = _dec_call(z, src_all, dst_all)

    lp = logits[:E].reshape(E // 128, 128)
    ln = logits[E:].reshape(E // 128, 128)

    loss, recon, kl = pl.pallas_call(
        _tc_loss_body,
        in_specs=[
            pl.BlockSpec((E // 128, 128), lambda: (0, 0)),
            pl.BlockSpec((E // 128, 128), lambda: (0, 0)),
            pl.BlockSpec((1, 1), lambda: (0, 0)),
        ],
        out_specs=[
            pl.BlockSpec((1, 1), lambda: (0, 0)),
            pl.BlockSpec((1, 1), lambda: (0, 0)),
            pl.BlockSpec((1, 1), lambda: (0, 0)),
        ],
        out_shape=[
            jax.ShapeDtypeStruct((1, 1), jnp.float32),
            jax.ShapeDtypeStruct((1, 1), jnp.float32),
            jax.ShapeDtypeStruct((1, 1), jnp.float32),
        ],
    )(lp, ln, klsum)

    return (loss.reshape(()),
            jax.lax.stop_gradient(recon.reshape(())),
            jax.lax.stop_gradient(kl.reshape(())))


# speedup vs baseline: 33.4090x; 1.9906x over previous
import jax
import jax.numpy as jnp
from jax import lax
from jax.experimental import pallas as pl
from jax.experimental.pallas import tpu as pltpu
from jax.experimental.pallas import tpu_sc as plsc

N = 10000
E = 320000
IN_DIM = 128
HID = 64
ZD = 32

NC, NS, L = 2, 16, 16
NW = NC * NS
EP = E // NW
EPD = 2 * E // NW
SUBK = 80
NSUBC = 5
CHE = SUBK * NSUBC
NCH_A = EP // CHE
NCH_D = EPD // CHE
NPAD = 10240
NPT = NPAD // NS
ZROWS = 160

RB = 1000
GRID = N // RB

_MESH = plsc.VectorSubcoreMesh(
    core_axis_name="c", subcore_axis_name="s", num_cores=NC, num_subcores=NS)
_SC_PARAMS = pltpu.CompilerParams(needs_layout_passes=False,
                                  use_tc_tiling_on_sc=False)


def _sc_deg_body(dst_hbm, out_hbm, dstbuf, countbuf):
    cid = lax.axis_index("c")
    sid = lax.axis_index("s")
    wid = sid * NC + cid
    zeros16 = jnp.zeros((L,), jnp.float32)
    ones16 = jnp.ones((L,), jnp.float32)

    def zb(i, c):
        countbuf[pl.ds(i * L, L)] = zeros16
        return c
    lax.fori_loop(0, N // L, zb, 0)

    pltpu.sync_copy(dst_hbm.at[pl.ds(wid * EP, EP)], dstbuf)

    def cb(i, c):
        idx = dstbuf[pl.ds(i * L, L)]
        plsc.addupdate_scatter(countbuf, [idx], ones16)
        return c
    lax.fori_loop(0, EP // L, cb, 0)

    for g in range(GRID):
        pltpu.sync_copy(countbuf.at[pl.ds(g * RB, RB)], out_hbm.at[g, wid])


_deg_call = pl.kernel(
    _sc_deg_body,
    out_type=jax.ShapeDtypeStruct((GRID, NW, RB), jnp.float32),
    mesh=_MESH,
    compiler_params=_SC_PARAMS,
    scratch_types=[
        pltpu.VMEM((EP,), jnp.int32),
        pltpu.VMEM((N,), jnp.float32),
    ],
)


def _sc_agg_body(tab_hbm, src_hbm, dst3_hbm, out_hbm, sidx, didx2, rows, zbuf,
                 acc, gsem, ssem):
    cid = lax.axis_index("c")
    sid = lax.axis_index("s")
    wid = sid * NC + cid
    zeros16 = jnp.zeros((L,), jnp.float32)

    def zb(i, c):
        for j in range(HID // L):
            zbuf[i, pl.ds(j * L, L)] = zeros16
        return c
    lax.fori_loop(0, ZROWS, zb, 0)
    for r in range(NPT // ZROWS):
        pltpu.sync_copy(zbuf, acc.at[pl.ds(sid * NPT + r * ZROWS, ZROWS)])
    plsc.subcore_barrier()

    pltpu.sync_copy(src_hbm.at[pl.ds(wid * EP, EP)], sidx)
    pltpu.sync_copy(dst3_hbm.at[wid], didx2)

    def start_gathers(c, b):
        for j in range(NSUBC):
            pltpu.async_copy(
                tab_hbm.at[sidx.at[pl.ds(c * CHE + j * SUBK, SUBK)]],
                rows.at[b].at[pl.ds(j * SUBK, SUBK)], gsem.at[b])

    start_gathers(0, 0)
    start_gathers(1, 1)

    def body(c, carry):
        b = lax.rem(c, 2)
        pltpu.make_async_copy(tab_hbm.at[pl.ds(0, CHE)], rows.at[b],
                              gsem.at[b]).wait()
        for j in range(NSUBC):
            pltpu.async_copy(rows.at[b].at[pl.ds(j * SUBK, SUBK)],
                             acc.at[didx2.at[c * NSUBC + j]], ssem.at[b],
                             add=True)
        pltpu.make_async_copy(tab_hbm.at[pl.ds(0, CHE)], rows.at[b],
                              ssem.at[b]).wait()

        @pl.when(c + 2 < NCH_A)
        def _():
            start_gathers(c + 2, b)
        return carry
    lax.fori_loop(0, NCH_A, body, 0)

    plsc.subcore_barrier()
    pltpu.sync_copy(acc.at[pl.ds(sid * NPT, NPT)],
                    out_hbm.at[cid, pl.ds(sid * NPT, NPT)])


_agg_call = pl.kernel(
    _sc_agg_body,
    out_type=jax.ShapeDtypeStruct((NC, NPAD, HID), jnp.float32),
    mesh=_MESH,
    compiler_params=_SC_PARAMS,
    scratch_types=[
        pltpu.VMEM((EP,), jnp.int32),
        pltpu.VMEM((NCH_A * NSUBC, SUBK), jnp.int32),
        pltpu.VMEM((2, CHE, HID), jnp.float32),
        pltpu.VMEM((ZROWS, HID), jnp.float32),
        pltpu.VMEM_SHARED((NPAD, HID), jnp.float32),
        pltpu.SemaphoreType.DMA((2,)),
        pltpu.SemaphoreType.DMA((2,)),
    ],
)


PTP = 24


def _sc_dec_body(z_hbm, src_hbm, dst_hbm, out_hbm, sidx, didx, zs, zd, lbuf,
                 ptmp, gsem):
    cid = lax.axis_index("c")
    sid = lax.axis_index("s")
    wid = sid * NC + cid
    iota = lax.iota(jnp.int32, L)
    zeros16 = jnp.zeros((L,), jnp.float32)

    pltpu.sync_copy(src_hbm.at[pl.ds(wid * EPD, EPD)], sidx)
    pltpu.sync_copy(dst_hbm.at[pl.ds(wid * EPD, EPD)], didx)

    def start_gathers(c, b):
        for j in range(NSUBC):
            off = c * CHE + j * SUBK
            pltpu.async_copy(z_hbm.at[sidx.at[pl.ds(off, SUBK)]],
                             zs.at[b].at[pl.ds(j * SUBK, SUBK)], gsem.at[b])
            pltpu.async_copy(z_hbm.at[didx.at[pl.ds(off, SUBK)]],
                             zd.at[b].at[pl.ds(j * SUBK, SUBK)], gsem.at[b])

    start_gathers(0, 0)
    start_gathers(1, 1)

    def body(c, carry):
        b = lax.rem(c, 2)
        pltpu.make_async_copy(z_hbm.at[pl.ds(0, CHE)], zs.at[b],
                              gsem.at[b]).wait()
        pltpu.make_async_copy(z_hbm.at[pl.ds(0, CHE)], zd.at[b],
                              gsem.at[b]).wait()

        def gb(g, cc):
            row0 = g * L
            for k in range(L):
                a1 = zs[b, row0 + k, pl.ds(0, L)]
                a2 = zs[b, row0 + k, pl.ds(L, L)]
                d1 = zd[b, row0 + k, pl.ds(0, L)]
                d2 = zd[b, row0 + k, pl.ds(L, L)]
                ptmp[k, pl.ds(0, L)] = a1 * d1 + a2 * d2
            q = zeros16
            for col in range(L):
                colv = jnp.full((L,), col, jnp.int32)
                q = q + plsc.load_gather(ptmp, [iota, colv])
            lbuf[pl.ds(c * CHE + g * L, L)] = q
            return cc
        lax.fori_loop(0, CHE // L, gb, 0)

        @pl.when(c + 2 < NCH_D)
        def _():
            start_gathers(c + 2, b)
        return carry
    lax.fori_loop(0, NCH_D, body, 0)

    pltpu.sync_copy(lbuf, out_hbm.at[pl.ds(wid * EPD, EPD)])


_dec_call = pl.kernel(
    _sc_dec_body,
    out_type=jax.ShapeDtypeStruct((2 * E,), jnp.float32),
    mesh=_MESH,
    compiler_params=_SC_PARAMS,
    scratch_types=[
        pltpu.VMEM((EPD,), jnp.int32),
        pltpu.VMEM((EPD,), jnp.int32),
        pltpu.VMEM((2, CHE, ZD), jnp.float32),
        pltpu.VMEM((2, CHE, ZD), jnp.float32),
        pltpu.VMEM((EPD,), jnp.float32),
        pltpu.VMEM((L, PTP), jnp.float32),
        pltpu.SemaphoreType.DMA((2,)),
    ],
)


def _tc_prep_body(counts_ref, x_ref, w1_ref, hs_ref, dinv_ref):
    deg = jnp.sum(counts_ref[0], axis=0) + 1.0
    dinv = lax.rsqrt(deg)
    h = jnp.dot(x_ref[...], w1_ref[...], preferred_element_type=jnp.float32)
    hs_ref[...] = h * dinv[:, None]
    dinv_ref[...] = dinv[:, None]


def _tc_h_body(t_ref, hs_ref, dinv_ref, b1_ref, out_ref):
    t = t_ref[0] + t_ref[1] + hs_ref[...]
    dinv = dinv_ref[...]
    h = jnp.maximum(t * dinv + b1_ref[...], 0.0)
    out_ref[...] = h * dinv


def _tc_z_body(t_ref, hs2_ref, dinv_ref, eps_ref, wmu_ref, bmu_ref, wlv_ref,
               blv_ref, z_ref, kl_ref):
    i = pl.program_id(0)
    agg = (t_ref[0] + t_ref[1] + hs2_ref[...]) * dinv_ref[...]
    mu = jnp.dot(agg, wmu_ref[...], preferred_element_type=jnp.float32) + bmu_ref[...]
    lv = jnp.dot(agg, wlv_ref[...], preferred_element_type=jnp.float32) + blv_ref[...]
    z_ref[...] = mu + eps_ref[...] * jnp.exp(0.5 * lv)
    klp = jnp.sum(1.0 + lv - mu * mu - jnp.exp(lv)).reshape(1, 1)

    @pl.when(i == 0)
    def _():
        kl_ref[...] = klp

    @pl.when(i > 0)
    def _():
        kl_ref[...] = kl_ref[...] + klp


def _tc_loss_body(lp_ref, ln_ref, kl_ref, loss_ref, recon_ref, klo_ref):
    lp = lp_ref[...]
    ln = ln_ref[...]
    sp_pos = jnp.maximum(lp, 0.0) - lp + jnp.log1p(jnp.exp(-jnp.abs(lp)))
    sp_neg = jnp.maximum(ln, 0.0) + jnp.log1p(jnp.exp(-jnp.abs(ln)))
    recon = ((jnp.sum(sp_pos) + jnp.sum(sp_neg)) / (2.0 * E)).reshape(1, 1)
    kl = -0.5 * kl_ref[...] / (N * ZD)
    loss_ref[...] = recon + kl
    recon_ref[...] = recon
    klo_ref[...] = kl


def kernel(x, edge_index, neg_edge_index, eps, W1, b1, Wmu, bmu, Wlv, blv):
    src = edge_index[0].astype(jnp.int32)
    dst = edge_index[1].astype(jnp.int32)
    dst3 = dst.reshape(NW, NCH_A * NSUBC, SUBK)

    counts = _deg_call(dst)

    hs1, dinv = pl.pallas_call(
        _tc_prep_body,
        grid=(GRID,),
        in_specs=[
            pl.BlockSpec((1, NW, RB), lambda i: (i, 0, 0)),
            pl.BlockSpec((RB, IN_DIM), lambda i: (i, 0)),
            pl.BlockSpec((IN_DIM, HID), lambda i: (0, 0)),
        ],
        out_specs=[
            pl.BlockSpec((RB, HID), lambda i: (i, 0)),
            pl.BlockSpec((RB, 1), lambda i: (i, 0)),
        ],
        out_shape=[
            jax.ShapeDtypeStruct((N, HID), jnp.float32),
            jax.ShapeDtypeStruct((N, 1), jnp.float32),
        ],
    )(counts, x, W1)

    t1 = _agg_call(hs1, src, dst3)

    hs2 = pl.pallas_call(
        _tc_h_body,
        grid=(GRID,),
        in_specs=[
            pl.BlockSpec((NC, RB, HID), lambda i: (0, i, 0)),
            pl.BlockSpec((RB, HID), lambda i: (i, 0)),
            pl.BlockSpec((RB, 1), lambda i: (i, 0)),
            pl.BlockSpec((1, HID), lambda i: (0, 0)),
        ],
        out_specs=pl.BlockSpec((RB, HID), lambda i: (i, 0)),
        out_shape=jax.ShapeDtypeStruct((N, HID), jnp.float32),
    )(t1, hs1, dinv, b1.reshape(1, HID))

    t2 = _agg_call(hs2, src, dst3)

    z, klsum = pl.pallas_call(
        _tc_z_body,
        grid=(GRID,),
        in_specs=[
            pl.BlockSpec((NC, RB, HID), lambda i: (0, i, 0)),
            pl.BlockSpec((RB, HID), lambda i: (i, 0)),
            pl.BlockSpec((RB, 1), lambda i: (i, 0)),
            pl.BlockSpec((RB, ZD), lambda i: (i, 0)),
            pl.BlockSpec((HID, ZD), lambda i: (0, 0)),
            pl.BlockSpec((1, ZD), lambda i: (0, 0)),
            pl.BlockSpec((HID, ZD), lambda i: (0, 0)),
            pl.BlockSpec((1, ZD), lambda i: (0, 0)),
        ],
        out_specs=[
            pl.BlockSpec((RB, ZD), lambda i: (i, 0)),
            pl.BlockSpec((1, 1), lambda i: (0, 0)),
        ],
        out_shape=[
            jax.ShapeDtypeStruct((N, ZD), jnp.float32),
            jax.ShapeDtypeStruct((1, 1), jnp.float32),
        ],
    )(t2, hs2, dinv, eps, Wmu, bmu.reshape(1, ZD), Wlv, blv.reshape(1, ZD))

    src_all = jnp.concatenate([src, neg_edge_index[0].astype(jnp.int32)])
    dst_all = jnp.concatenate([dst, neg_edge_index[1].astype(jnp.int32)])
    logits = _dec_call(z, src_all, dst_all)

    lp = logits[:E].reshape(E // 128, 128)
    ln = logits[E:].reshape(E // 128, 128)

    loss, recon, kl = pl.pallas_call(
        _tc_loss_body,
        in_specs=[
            pl.BlockSpec((E // 128, 128), lambda: (0, 0)),
            pl.BlockSpec((E // 128, 128), lambda: (0, 0)),
            pl.BlockSpec((1, 1), lambda: (0, 0)),
        ],
        out_specs=[
            pl.BlockSpec((1, 1), lambda: (0, 0)),
            pl.BlockSpec((1, 1), lambda: (0, 0)),
            pl.BlockSpec((1, 1), lambda: (0, 0)),
        ],
        out_shape=[
            jax.ShapeDtypeStruct((1, 1), jnp.float32),
            jax.ShapeDtypeStruct((1, 1), jnp.float32),
            jax.ShapeDtypeStruct((1, 1), jnp.float32),
        ],
    )(lp, ln, klsum)

    return (loss.reshape(()),
            jax.lax.stop_gradient(recon.reshape(())),
            jax.lax.stop_gradient(kl.reshape(())))
